# Initial kernel scaffold; baseline (speedup 1.0000x reference)
import jax
import jax.numpy as jnp
from jax.experimental import pallas as pl


def kernel(x, X, W_mlp, W0, W1, W2, W_proj):
    raise NotImplementedError("write your pallas kernel here")


# speedup vs baseline: 50.0648x; 50.0648x over previous
import jax
import jax.numpy as jnp
from jax.experimental import pallas as pl

DIM = 1024
N_EXP = 16
DIM_GCN = 256
THRESH = 0.8
NTOK = 256


def _moegnn_body(x_ref, X_ref, Wm_ref, W0_ref, W1_ref, W2_ref, Wp_ref, out_ref):
    f32 = jnp.float32
    Wm = Wm_ref[...]
    Xc = X_ref[...]
    W0 = W0_ref[...]
    W1 = W1_ref[...]

    expc = jnp.maximum(
        jax.lax.dot_general(Wm, Xc, (((1,), (0,)), ((), ())),
                            preferred_element_type=f32), 0.0)

    nrm2 = jnp.sum(expc * expc, axis=0, keepdims=True)
    nrm = jnp.maximum(jnp.sqrt(nrm2), 1e-8)
    G = jax.lax.dot_general(expc, expc, (((0,), (0,)), ((), ())),
                            preferred_element_type=f32)
    ri = jax.lax.broadcasted_iota(jnp.int32, (N_EXP, N_EXP), 0)
    ci = jax.lax.broadcasted_iota(jnp.int32, (N_EXP, N_EXP), 1)
    denom = nrm * jnp.ones((N_EXP, 1), f32)
    denomT = nrm.reshape(N_EXP, 1) * jnp.ones((1, N_EXP), f32)
    cos = G / (denom * denomT)
    ind = (cos > THRESH).astype(f32)
    lower = jnp.where(ri > ci, ind, 0.0)
    upper = jnp.where(ri < ci, ind, 0.0)

    dinv_col = jax.lax.rsqrt(2.0 + jnp.sum(lower, axis=1, keepdims=True))
    dinv_row = jax.lax.rsqrt(2.0 + jnp.sum(upper, axis=0, keepdims=True))
    eye = jnp.where(ri == ci, 1.0, 0.0)
    A = dinv_col * dinv_row * (lower + eye)

    EW0 = jax.lax.dot_general(expc, W0, (((0,), (0,)), ((), ())),
                              preferred_element_type=f32)
    C0 = jnp.dot(A, EW0, preferred_element_type=f32)
    C1 = jnp.dot(jnp.dot(A, C0, preferred_element_type=f32), W1,
                 preferred_element_type=f32)
    b = jnp.dot(A, dinv_col, preferred_element_type=f32) + dinv_col
    v_row = jax.lax.dot_general(Wp_ref[...], W2_ref[...],
                                (((0,), (1,)), ((), ())),
                                preferred_element_type=f32)

    XF = jnp.maximum(
        jax.lax.dot_general(x_ref[...], Wm, (((1,), (1,)), ((), ())),
                            preferred_element_type=f32), 0.0)
    U0 = jnp.dot(XF, W0, preferred_element_type=f32)
    U1 = jnp.dot(U0, W1, preferred_element_type=f32)

    cols = []
    for i in range(N_EXP):
        bi = jax.lax.slice(b, (i, 0), (i + 1, 1))
        c1i = jax.lax.slice(C1, (i, 0), (i + 1, DIM_GCN))
        hi = jnp.maximum(U1 * bi + c1i, 0.0)
        cols.append(jnp.sum(hi * v_row, axis=1, keepdims=True))
    R = jnp.concatenate(cols, axis=1)

    t_term = jnp.sum(jnp.maximum(U1, 0.0) * v_row, axis=1, keepdims=True)
    S = jax.lax.dot_general(R, A, (((1,), (1,)), ((), ())),
                            preferred_element_type=f32)
    S = S + t_term * dinv_row

    m = jnp.max(S, axis=1, keepdims=True)
    e = jnp.exp(S - m)
    out_ref[...] = e / jnp.sum(e, axis=1, keepdims=True)


def kernel(x, X, W_mlp, W0, W1, W2, W_proj):
    ori_shape = x.shape[:-1]
    x2 = x.reshape(-1, DIM)
    out = pl.pallas_call(
        _moegnn_body,
        out_shape=jax.ShapeDtypeStruct((NTOK, N_EXP), jnp.float32),
    )(x2, X, W_mlp, W0, W1, W2, W_proj)
    return out.reshape(*ori_shape, N_EXP)
